# baseline (speedup 1.0000x reference)
import dataclasses
import functools

import jax
import jax.numpy as jnp
from jax import lax
from jax.experimental import pallas as pl
from jax.experimental.pallas import tpu as pltpu
from jax.experimental.pallas import tpu_sc as plsc

MAX_NORM = 1.0 - 0.0001
MAX_NORM_SQ = MAX_NORM * MAX_NORM
LANES = 16
NUM_CORES = 2
NUM_SUBCORES = 16
NUM_WORKERS = NUM_CORES * NUM_SUBCORES


@functools.partial(jax.jit, static_argnames=("b", "m", "chunk"))
def _sc_embed(x_flat, w_view, *, b, m, chunk):
    per_w = b // NUM_WORKERS
    n_chunks = per_w // chunk

    mesh = plsc.VectorSubcoreMesh(core_axis_name="c", subcore_axis_name="s")

    cparams = pltpu.CompilerParams()
    if "needs_layout_passes" in pltpu.CompilerParams.__dataclass_fields__:
        cparams = dataclasses.replace(cparams, needs_layout_passes=False)
    if "use_tc_tiling_on_sc" in pltpu.CompilerParams.__dataclass_fields__:
        cparams = dataclasses.replace(cparams, use_tc_tiling_on_sc=True)

    @functools.partial(
        pl.kernel,
        out_type=jax.ShapeDtypeStruct((b, m), jnp.float32),
        mesh=mesh,
        compiler_params=cparams,
        scratch_types=[
            pltpu.VMEM((chunk,), jnp.int32),
            pltpu.VMEM((chunk,), jnp.int32),
            pltpu.VMEM((chunk, 2 * m), jnp.float32),
            pltpu.VMEM((chunk, m), jnp.float32),
            pltpu.SemaphoreType.DMA,
        ],
    )
    def body(x_hbm, w_hbm, out_hbm, idx_v, idx_g, rows_v, stage_v, sem):
        wid = lax.axis_index("s") * NUM_CORES + lax.axis_index("c")
        base = wid * per_w
        lane_iota = lax.iota(jnp.int32, LANES)

        @pl.loop(0, n_chunks)
        def _(k):
            off = base + k * chunk
            pltpu.sync_copy(x_hbm.at[pl.ds(off, chunk)], idx_v)

            @pl.loop(0, chunk, step=LANES)
            def _(c):
                sl = pl.ds(c, LANES)
                idx_g[sl] = idx_v[sl] >> 1

            pltpu.async_copy(w_hbm.at[idx_g], rows_v, sem).wait()

            @pl.loop(0, chunk, step=LANES)
            def _(r0):
                rows16 = lane_iota + r0
                colbase = (idx_v[pl.ds(r0, LANES)] & 1) * m
                acc = jnp.zeros((LANES,), jnp.float32)
                for j in range(m):
                    v = plsc.load_gather(rows_v, [rows16, colbase + j])
                    plsc.store_scatter(
                        stage_v, [rows16, jnp.full((LANES,), j, jnp.int32)], v
                    )
                    acc = acc + v * v

                @pl.when(jnp.max(acc) > MAX_NORM_SQ)
                def _():
                    bits = lax.bitcast_convert_type(acc, jnp.int32)
                    y = lax.bitcast_convert_type(
                        0x5F3759DF - (bits >> 1), jnp.float32
                    )
                    for _ in range(3):
                        y = y * (1.5 - 0.5 * acc * y * y)
                    norm = acc * y
                    scale = jnp.where(
                        acc > MAX_NORM_SQ,
                        MAX_NORM / (norm + 1e-7),
                        jnp.float32(1.0),
                    )
                    for j in range(m):
                        col = jnp.full((LANES,), j, jnp.int32)
                        v = plsc.load_gather(stage_v, [rows16, col]) * scale
                        plsc.store_scatter(stage_v, [rows16, col], v)

            pltpu.sync_copy(stage_v, out_hbm.at[pl.ds(off, chunk)])

    return body(x_flat, w_view)


def kernel(x, weight):
    bsz, hist = x.shape
    n, m = weight.shape
    b = bsz * hist
    x_flat = x.reshape(b).astype(jnp.int32)
    w_view = weight.reshape(n // 2, 2 * m)
    out = _sc_embed(x_flat, w_view, b=b, m=m, chunk=256)
    return out.reshape(bsz, hist, m)


# speedup vs baseline: 1.3454x; 1.3454x over previous
import dataclasses
import functools

import jax
import jax.numpy as jnp
from jax import lax
from jax.experimental import pallas as pl
from jax.experimental.pallas import tpu as pltpu
from jax.experimental.pallas import tpu_sc as plsc

MAX_NORM = 1.0 - 0.0001
MAX_NORM_SQ = MAX_NORM * MAX_NORM
LANES = 16
GATHER_W = 128
NUM_CORES = 2
NUM_SUBCORES = 16
NUM_WORKERS = NUM_CORES * NUM_SUBCORES


def _pad_rows(wt, *, n, m, vblk):
    def body(x_ref, o_ref):
        o_ref[:, :m] = x_ref[...].T
        o_ref[:, m:] = jnp.zeros((vblk, GATHER_W - m), jnp.float32)

    return pl.pallas_call(
        body,
        grid=((n + vblk - 1) // vblk,),
        in_specs=[pl.BlockSpec((m, vblk), lambda i: (0, i))],
        out_specs=pl.BlockSpec((vblk, GATHER_W), lambda i: (i, 0)),
        out_shape=jax.ShapeDtypeStruct((n, GATHER_W), jnp.float32),
    )(wt)


@functools.partial(jax.jit, static_argnames=("bsz", "hist", "m", "rows_blk"))
def _sc_embed(x_flat, w_pad, *, bsz, hist, m, rows_blk):
    b = bsz * hist
    per_w = b // NUM_WORKERS
    chunk = rows_blk * hist
    n_chunks = per_w // chunk
    n_sub = m // LANES
    b_per_w = bsz // NUM_WORKERS

    mesh = plsc.VectorSubcoreMesh(core_axis_name="c", subcore_axis_name="s")

    cparams = pltpu.CompilerParams()
    if "needs_layout_passes" in pltpu.CompilerParams.__dataclass_fields__:
        cparams = dataclasses.replace(cparams, needs_layout_passes=False)
    if "use_tc_tiling_on_sc" in pltpu.CompilerParams.__dataclass_fields__:
        cparams = dataclasses.replace(cparams, use_tc_tiling_on_sc=True)

    @functools.partial(
        pl.kernel,
        out_type=jax.ShapeDtypeStruct((b, m), jnp.float32),
        mesh=mesh,
        compiler_params=cparams,
        scratch_types=[
            pltpu.VMEM((chunk,), jnp.int32),
            pltpu.VMEM((chunk, GATHER_W), jnp.float32),
            pltpu.VMEM((chunk, m), jnp.float32),
            pltpu.SemaphoreType.DMA,
        ],
    )
    def body(x_hbm, w_hbm, out_hbm, idx_v, gbuf, stage, sem):
        wid = lax.axis_index("s") * NUM_CORES + lax.axis_index("c")
        base = wid * per_w

        @pl.loop(0, n_chunks)
        def _(k):
            off = base + k * chunk
            pltpu.sync_copy(x_hbm.at[pl.ds(off, chunk)], idx_v)
            pltpu.async_copy(w_hbm.at[idx_v], gbuf, sem).wait()

            def repack_row(q, gmax):
                p = jnp.zeros((LANES,), jnp.float32)
                for j in range(n_sub):
                    v = gbuf[q, pl.ds(j * LANES, LANES)]
                    stage[q, pl.ds(j * LANES, LANES)] = v
                    p = p + v * v
                return jnp.maximum(gmax, p)

            gmax = lax.fori_loop(
                0, chunk, repack_row, jnp.zeros((LANES,), jnp.float32)
            )
            bound = jnp.sum(gmax)

            @pl.when(bound > MAX_NORM_SQ)
            def _():
                def fix_row(q, carry):
                    vs = [
                        stage[q, pl.ds(j * LANES, LANES)]
                        for j in range(n_sub)
                    ]
                    p = jnp.zeros((LANES,), jnp.float32)
                    for v in vs:
                        p = p + v * v
                    s2 = jnp.sum(p)
                    s2v = lax.broadcast(s2, (LANES,))
                    bits = lax.bitcast_convert_type(s2v, jnp.int32)
                    y = lax.bitcast_convert_type(
                        0x5F3759DF - (bits >> 1), jnp.float32
                    )
                    for _ in range(3):
                        y = y * (1.5 - 0.5 * s2v * y * y)
                    norm = s2v * y
                    scale = jnp.where(
                        s2v > MAX_NORM_SQ,
                        MAX_NORM / (norm + 1e-7),
                        jnp.float32(1.0),
                    )
                    for j, v in enumerate(vs):
                        stage[q, pl.ds(j * LANES, LANES)] = v * scale
                    return carry

                lax.fori_loop(0, chunk, fix_row, 0)

            pltpu.sync_copy(stage, out_hbm.at[pl.ds(off, chunk)])

    return body(x_flat, w_pad)


def kernel(x, weight):
    bsz, hist = x.shape
    n, m = weight.shape
    x_flat = x.reshape(bsz * hist).astype(jnp.int32)
    w_pad = _pad_rows(weight.T, n=n, m=m, vblk=1024)
    out = _sc_embed(x_flat, w_pad, bsz=bsz, hist=hist, m=m, rows_blk=8)
    return out.reshape(bsz, hist, m)


# speedup vs baseline: 2.1538x; 1.6008x over previous
import dataclasses
import functools

import jax
import jax.numpy as jnp
from jax import lax
from jax.experimental import pallas as pl
from jax.experimental.pallas import tpu as pltpu
from jax.experimental.pallas import tpu_sc as plsc

MAX_NORM = 1.0 - 0.0001
MAX_NORM_SQ = MAX_NORM * MAX_NORM
LANES = 16
GATHER_W = 128
NUM_CORES = 2
NUM_SUBCORES = 16
NUM_WORKERS = NUM_CORES * NUM_SUBCORES


def _pad_rows(wt, *, n, m, vblk):
    def body(x_ref, o_ref):
        o_ref[:, :m] = x_ref[...].T
        o_ref[:, m:] = jnp.zeros((vblk, GATHER_W - m), jnp.float32)

    return pl.pallas_call(
        body,
        grid=((n + vblk - 1) // vblk,),
        in_specs=[pl.BlockSpec((m, vblk), lambda i: (0, i))],
        out_specs=pl.BlockSpec((vblk, GATHER_W), lambda i: (i, 0)),
        out_shape=jax.ShapeDtypeStruct((n, GATHER_W), jnp.float32),
    )(wt)


@functools.partial(jax.jit, static_argnames=("bsz", "hist", "m", "rows_blk"))
def _sc_embed(x_flat, w_pad, *, bsz, hist, m, rows_blk):
    b = bsz * hist
    per_w = b // NUM_WORKERS
    chunk = rows_blk * hist
    n_chunks = per_w // chunk
    n_sub = m // LANES
    b_per_w = bsz // NUM_WORKERS

    mesh = plsc.VectorSubcoreMesh(core_axis_name="c", subcore_axis_name="s")

    cparams = pltpu.CompilerParams()
    if "needs_layout_passes" in pltpu.CompilerParams.__dataclass_fields__:
        cparams = dataclasses.replace(cparams, needs_layout_passes=False)
    if "use_tc_tiling_on_sc" in pltpu.CompilerParams.__dataclass_fields__:
        cparams = dataclasses.replace(cparams, use_tc_tiling_on_sc=True)

    @functools.partial(
        pl.kernel,
        out_type=jax.ShapeDtypeStruct((b, m), jnp.float32),
        mesh=mesh,
        compiler_params=cparams,
        scratch_types=[
            pltpu.VMEM((chunk,), jnp.int32),
            pltpu.VMEM((chunk, GATHER_W), jnp.float32),
            pltpu.VMEM((chunk, m), jnp.float32),
            pltpu.SemaphoreType.DMA,
        ],
    )
    def body(x_hbm, w_hbm, out_hbm, idx_v, gbuf, stage, sem):
        wid = lax.axis_index("s") * NUM_CORES + lax.axis_index("c")
        base = wid * per_w

        @pl.loop(0, n_chunks)
        def _(k):
            off = base + k * chunk
            pltpu.sync_copy(x_hbm.at[pl.ds(off, chunk)], idx_v)
            pltpu.async_copy(w_hbm.at[idx_v], gbuf, sem).wait()

            def repack_row(q, gmax):
                p = jnp.zeros((LANES,), jnp.float32)
                for j in range(n_sub):
                    v = gbuf[q, pl.ds(j * LANES, LANES)]
                    stage[q, pl.ds(j * LANES, LANES)] = v
                    p = p + v * v
                return jnp.maximum(gmax, p)

            gmax = lax.fori_loop(
                0, chunk, repack_row, jnp.zeros((LANES,), jnp.float32)
            )
            bound = jnp.sum(gmax)

            @pl.when(bound > MAX_NORM_SQ)
            def _():
                def fix_row(q, carry):
                    vs = [
                        stage[q, pl.ds(j * LANES, LANES)]
                        for j in range(n_sub)
                    ]
                    p = jnp.zeros((LANES,), jnp.float32)
                    for v in vs:
                        p = p + v * v
                    s2 = jnp.sum(p)
                    s2v = lax.broadcast(s2, (LANES,))
                    bits = lax.bitcast_convert_type(s2v, jnp.int32)
                    y = lax.bitcast_convert_type(
                        0x5F3759DF - (bits >> 1), jnp.float32
                    )
                    for _ in range(3):
                        y = y * (1.5 - 0.5 * s2v * y * y)
                    norm = s2v * y
                    scale = jnp.where(
                        s2v > MAX_NORM_SQ,
                        MAX_NORM / (norm + 1e-7),
                        jnp.float32(1.0),
                    )
                    for j, v in enumerate(vs):
                        stage[q, pl.ds(j * LANES, LANES)] = v * scale
                    return carry

                lax.fori_loop(0, chunk, fix_row, 0)

            pltpu.sync_copy(stage, out_hbm.at[pl.ds(off, chunk)])

    return body(x_flat, w_pad)


def kernel(x, weight):
    bsz, hist = x.shape
    n, m = weight.shape
    x_flat = x.reshape(bsz * hist).astype(jnp.int32)
    w_pad = _pad_rows(weight.T, n=n, m=m, vblk=4096)
    out = _sc_embed(x_flat, w_pad, bsz=bsz, hist=hist, m=m, rows_blk=8)
    return out.reshape(bsz, hist, m)


# speedup vs baseline: 2.4550x; 1.1398x over previous
import dataclasses
import functools

import jax
import jax.numpy as jnp
from jax import lax
from jax.experimental import pallas as pl
from jax.experimental.pallas import tpu as pltpu
from jax.experimental.pallas import tpu_sc as plsc

MAX_NORM = 1.0 - 0.0001
MAX_NORM_SQ = MAX_NORM * MAX_NORM
LANES = 16
GATHER_W = 128
NUM_CORES = 2
NUM_SUBCORES = 16
NUM_WORKERS = NUM_CORES * NUM_SUBCORES


def _pad_rows(wt, *, n, m, vblk):
    def body(x_ref, o_ref):
        o_ref[:, :m] = x_ref[...].T
        o_ref[:, m:] = jnp.zeros((vblk, GATHER_W - m), jnp.float32)

    return pl.pallas_call(
        body,
        grid=((n + vblk - 1) // vblk,),
        in_specs=[pl.BlockSpec((m, vblk), lambda i: (0, i))],
        out_specs=pl.BlockSpec((vblk, GATHER_W), lambda i: (i, 0)),
        out_shape=jax.ShapeDtypeStruct((n, GATHER_W), jnp.float32),
    )(wt)


@functools.partial(jax.jit, static_argnames=("bsz", "hist", "m", "rows_blk"))
def _sc_embed(x_flat, w_pad, *, bsz, hist, m, rows_blk):
    b = bsz * hist
    per_w = b // NUM_WORKERS
    chunk = rows_blk * hist
    n_chunks = per_w // chunk
    n_sub = m // LANES
    b_per_w = bsz // NUM_WORKERS

    mesh = plsc.VectorSubcoreMesh(core_axis_name="c", subcore_axis_name="s")

    cparams = pltpu.CompilerParams()
    if "needs_layout_passes" in pltpu.CompilerParams.__dataclass_fields__:
        cparams = dataclasses.replace(cparams, needs_layout_passes=False)
    if "use_tc_tiling_on_sc" in pltpu.CompilerParams.__dataclass_fields__:
        cparams = dataclasses.replace(cparams, use_tc_tiling_on_sc=True)

    @functools.partial(
        pl.kernel,
        out_type=jax.ShapeDtypeStruct((b, m), jnp.float32),
        mesh=mesh,
        compiler_params=cparams,
        scratch_types=[
            pltpu.VMEM((chunk,), jnp.int32),
            pltpu.VMEM((chunk, GATHER_W), jnp.float32),
            pltpu.VMEM((chunk, m), jnp.float32),
            pltpu.SemaphoreType.DMA,
        ],
    )
    def body(x_hbm, w_hbm, out_hbm, idx_v, gbuf, stage, sem):
        wid = lax.axis_index("s") * NUM_CORES + lax.axis_index("c")
        base = wid * per_w

        @pl.loop(0, n_chunks)
        def _(k):
            off = base + k * chunk
            pltpu.sync_copy(x_hbm.at[pl.ds(off, chunk)], idx_v)
            pltpu.async_copy(w_hbm.at[idx_v], gbuf, sem).wait()

            def repack_row(q, gmax):
                p = jnp.zeros((LANES,), jnp.float32)
                for j in range(n_sub):
                    v = gbuf[q, pl.ds(j * LANES, LANES)]
                    stage[q, pl.ds(j * LANES, LANES)] = v
                    p = p + v * v
                return jnp.maximum(gmax, p)

            gmax = lax.fori_loop(
                0, chunk, repack_row, jnp.zeros((LANES,), jnp.float32)
            )
            bound = jnp.sum(gmax)

            @pl.when(bound > MAX_NORM_SQ)
            def _():
                def fix_row(q, carry):
                    vs = [
                        stage[q, pl.ds(j * LANES, LANES)]
                        for j in range(n_sub)
                    ]
                    p = jnp.zeros((LANES,), jnp.float32)
                    for v in vs:
                        p = p + v * v
                    s2 = jnp.sum(p)
                    s2v = lax.broadcast(s2, (LANES,))
                    bits = lax.bitcast_convert_type(s2v, jnp.int32)
                    y = lax.bitcast_convert_type(
                        0x5F3759DF - (bits >> 1), jnp.float32
                    )
                    for _ in range(3):
                        y = y * (1.5 - 0.5 * s2v * y * y)
                    norm = s2v * y
                    scale = jnp.where(
                        s2v > MAX_NORM_SQ,
                        MAX_NORM / (norm + 1e-7),
                        jnp.float32(1.0),
                    )
                    for j, v in enumerate(vs):
                        stage[q, pl.ds(j * LANES, LANES)] = v * scale
                    return carry

                lax.fori_loop(0, chunk, fix_row, 0)

            pltpu.sync_copy(stage, out_hbm.at[pl.ds(off, chunk)])

    return body(x_flat, w_pad)


def kernel(x, weight):
    bsz, hist = x.shape
    n, m = weight.shape
    x_flat = x.reshape(bsz * hist).astype(jnp.int32)
    w_pad = _pad_rows(weight.T, n=n, m=m, vblk=8192)
    out = _sc_embed(x_flat, w_pad, bsz=bsz, hist=hist, m=m, rows_blk=8)
    return out.reshape(bsz, hist, m)


# speedup vs baseline: 2.5360x; 1.0330x over previous
import dataclasses
import functools

import jax
import jax.numpy as jnp
from jax import lax
from jax.experimental import pallas as pl
from jax.experimental.pallas import tpu as pltpu
from jax.experimental.pallas import tpu_sc as plsc

MAX_NORM = 1.0 - 0.0001
MAX_NORM_SQ = MAX_NORM * MAX_NORM
LANES = 16
GATHER_W = 128
NUM_CORES = 2
NUM_SUBCORES = 16
NUM_WORKERS = NUM_CORES * NUM_SUBCORES


def _pad_rows(wt, *, n, m, vblk):
    def body(x_ref, o_ref):
        o_ref[:, :m] = x_ref[...].T
        o_ref[:, m:] = jnp.zeros((vblk, GATHER_W - m), jnp.float32)

    return pl.pallas_call(
        body,
        grid=((n + vblk - 1) // vblk,),
        in_specs=[pl.BlockSpec((m, vblk), lambda i: (0, i))],
        out_specs=pl.BlockSpec((vblk, GATHER_W), lambda i: (i, 0)),
        out_shape=jax.ShapeDtypeStruct((n, GATHER_W), jnp.float32),
    )(wt)


@functools.partial(jax.jit, static_argnames=("bsz", "hist", "m", "rows_blk"))
def _sc_embed(x_flat, w_pad, *, bsz, hist, m, rows_blk):
    b = bsz * hist
    per_w = b // NUM_WORKERS
    chunk = rows_blk * hist
    n_chunks = per_w // chunk
    n_sub = m // LANES
    b_per_w = bsz // NUM_WORKERS

    mesh = plsc.VectorSubcoreMesh(core_axis_name="c", subcore_axis_name="s")

    cparams = pltpu.CompilerParams()
    if "needs_layout_passes" in pltpu.CompilerParams.__dataclass_fields__:
        cparams = dataclasses.replace(cparams, needs_layout_passes=False)
    if "use_tc_tiling_on_sc" in pltpu.CompilerParams.__dataclass_fields__:
        cparams = dataclasses.replace(cparams, use_tc_tiling_on_sc=True)

    @functools.partial(
        pl.kernel,
        out_type=jax.ShapeDtypeStruct((b, m), jnp.float32),
        mesh=mesh,
        compiler_params=cparams,
        scratch_types=[
            pltpu.VMEM((chunk,), jnp.int32),
            pltpu.VMEM((chunk, GATHER_W), jnp.float32),
            pltpu.VMEM((chunk, m), jnp.float32),
            pltpu.SemaphoreType.DMA,
        ],
    )
    def body(x_hbm, w_hbm, out_hbm, idx_v, gbuf, stage, sem):
        wid = lax.axis_index("s") * NUM_CORES + lax.axis_index("c")
        base = wid * per_w

        @pl.loop(0, n_chunks)
        def _(k):
            off = base + k * chunk
            pltpu.sync_copy(x_hbm.at[pl.ds(off, chunk)], idx_v)
            pltpu.async_copy(w_hbm.at[idx_v], gbuf, sem).wait()

            def repack_row(q, gmax):
                p = jnp.zeros((LANES,), jnp.float32)
                for j in range(n_sub):
                    v = gbuf[q, pl.ds(j * LANES, LANES)]
                    stage[q, pl.ds(j * LANES, LANES)] = v
                    p = p + v * v
                return jnp.maximum(gmax, p)

            gmax = lax.fori_loop(
                0, chunk, repack_row, jnp.zeros((LANES,), jnp.float32)
            )
            bound = jnp.sum(gmax)

            @pl.when(bound > MAX_NORM_SQ)
            def _():
                def fix_row(q, carry):
                    vs = [
                        stage[q, pl.ds(j * LANES, LANES)]
                        for j in range(n_sub)
                    ]
                    p = jnp.zeros((LANES,), jnp.float32)
                    for v in vs:
                        p = p + v * v
                    s2 = jnp.sum(p)
                    s2v = lax.broadcast(s2, (LANES,))
                    bits = lax.bitcast_convert_type(s2v, jnp.int32)
                    y = lax.bitcast_convert_type(
                        0x5F3759DF - (bits >> 1), jnp.float32
                    )
                    for _ in range(3):
                        y = y * (1.5 - 0.5 * s2v * y * y)
                    norm = s2v * y
                    scale = jnp.where(
                        s2v > MAX_NORM_SQ,
                        MAX_NORM / (norm + 1e-7),
                        jnp.float32(1.0),
                    )
                    for j, v in enumerate(vs):
                        stage[q, pl.ds(j * LANES, LANES)] = v * scale
                    return carry

                lax.fori_loop(0, chunk, fix_row, 0)

            pltpu.sync_copy(stage, out_hbm.at[pl.ds(off, chunk)])

    return body(x_flat, w_pad)


def kernel(x, weight):
    bsz, hist = x.shape
    n, m = weight.shape
    x_flat = x.reshape(bsz * hist).astype(jnp.int32)
    w_pad = _pad_rows(weight.T, n=n, m=m, vblk=16384)
    out = _sc_embed(x_flat, w_pad, bsz=bsz, hist=hist, m=m, rows_blk=8)
    return out.reshape(bsz, hist, m)


# speedup vs baseline: 2.6334x; 1.0384x over previous
import dataclasses
import functools

import jax
import jax.numpy as jnp
from jax import lax
from jax.experimental import pallas as pl
from jax.experimental.pallas import tpu as pltpu
from jax.experimental.pallas import tpu_sc as plsc

MAX_NORM = 1.0 - 0.0001
MAX_NORM_SQ = MAX_NORM * MAX_NORM
LANES = 16
GATHER_W = 128
NUM_CORES = 2
NUM_SUBCORES = 16
NUM_WORKERS = NUM_CORES * NUM_SUBCORES


def _pad_rows(wt, *, n, m, vblk):
    def body(x_ref, o_ref):
        o_ref[:, :m] = x_ref[...].T
        o_ref[:, m:] = jnp.zeros((vblk, GATHER_W - m), jnp.float32)

    return pl.pallas_call(
        body,
        grid=((n + vblk - 1) // vblk,),
        in_specs=[pl.BlockSpec((m, vblk), lambda i: (0, i))],
        out_specs=pl.BlockSpec((vblk, GATHER_W), lambda i: (i, 0)),
        out_shape=jax.ShapeDtypeStruct((n, GATHER_W), jnp.float32),
    )(wt)


@functools.partial(jax.jit, static_argnames=("bsz", "hist", "m", "rows_blk"))
def _sc_embed(x_flat, w_pad, *, bsz, hist, m, rows_blk):
    b = bsz * hist
    per_w = b // NUM_WORKERS
    chunk = rows_blk * hist
    n_chunks = per_w // chunk
    n_sub = m // LANES

    mesh = plsc.VectorSubcoreMesh(core_axis_name="c", subcore_axis_name="s")

    cparams = pltpu.CompilerParams()
    if "needs_layout_passes" in pltpu.CompilerParams.__dataclass_fields__:
        cparams = dataclasses.replace(cparams, needs_layout_passes=False)
    if "use_tc_tiling_on_sc" in pltpu.CompilerParams.__dataclass_fields__:
        cparams = dataclasses.replace(cparams, use_tc_tiling_on_sc=True)

    @functools.partial(
        pl.kernel,
        out_type=jax.ShapeDtypeStruct((b, m), jnp.float32),
        mesh=mesh,
        compiler_params=cparams,
        scratch_types=[
            pltpu.VMEM((chunk,), jnp.int32),
            pltpu.VMEM((chunk,), jnp.int32),
            pltpu.VMEM((chunk, GATHER_W), jnp.float32),
            pltpu.VMEM((chunk, GATHER_W), jnp.float32),
            pltpu.VMEM((chunk, m), jnp.float32),
            pltpu.SemaphoreType.DMA,
            pltpu.SemaphoreType.DMA,
        ],
    )
    def body(
        x_hbm, w_hbm, out_hbm, idx_a, idx_b, gbuf_a, gbuf_b, stage, sem0, sem1
    ):
        wid = lax.axis_index("s") * NUM_CORES + lax.axis_index("c")
        base = wid * per_w
        idxs = (idx_a, idx_b)
        gbufs = (gbuf_a, gbuf_b)
        sems = (sem0, sem1)

        def fetch(k, buf):
            off = base + k * chunk
            pltpu.sync_copy(x_hbm.at[pl.ds(off, chunk)], idxs[buf])
            pltpu.async_copy(w_hbm.at[idxs[buf]], gbufs[buf], sems[buf])

        def process(k, phase):
            off = base + k * chunk
            gbuf = gbufs[phase]
            pltpu.make_async_copy(
                w_hbm.at[pl.ds(0, chunk)], gbuf, sems[phase]
            ).wait()

            @pl.when(k + 1 < n_chunks)
            def _():
                fetch(k + 1, 1 - phase)

            def repack_row(q, gmax):
                p = jnp.zeros((LANES,), jnp.float32)
                for j in range(n_sub):
                    v = gbuf[q, pl.ds(j * LANES, LANES)]
                    stage[q, pl.ds(j * LANES, LANES)] = v
                    p = p + v * v
                return jnp.maximum(gmax, p)

            gmax = lax.fori_loop(
                0, chunk, repack_row, jnp.zeros((LANES,), jnp.float32)
            )
            bound = jnp.sum(gmax)

            @pl.when(bound > MAX_NORM_SQ)
            def _():
                def fix_row(q, carry):
                    vs = [
                        stage[q, pl.ds(j * LANES, LANES)]
                        for j in range(n_sub)
                    ]
                    p = jnp.zeros((LANES,), jnp.float32)
                    for v in vs:
                        p = p + v * v
                    s2 = jnp.sum(p)
                    s2v = lax.broadcast(s2, (LANES,))
                    bits = lax.bitcast_convert_type(s2v, jnp.int32)
                    y = lax.bitcast_convert_type(
                        0x5F3759DF - (bits >> 1), jnp.float32
                    )
                    for _ in range(3):
                        y = y * (1.5 - 0.5 * s2v * y * y)
                    norm = s2v * y
                    scale = jnp.where(
                        s2v > MAX_NORM_SQ,
                        MAX_NORM / (norm + 1e-7),
                        jnp.float32(1.0),
                    )
                    for j, v in enumerate(vs):
                        stage[q, pl.ds(j * LANES, LANES)] = v * scale
                    return carry

                lax.fori_loop(0, chunk, fix_row, 0)

            pltpu.sync_copy(stage, out_hbm.at[pl.ds(off, chunk)])

        fetch(0, 0)

        @pl.loop(0, n_chunks // 2)
        def _(kk):
            process(kk * 2, 0)
            process(kk * 2 + 1, 1)

    return body(x_flat, w_pad)


def kernel(x, weight):
    bsz, hist = x.shape
    n, m = weight.shape
    x_flat = x.reshape(bsz * hist).astype(jnp.int32)
    w_pad = _pad_rows(weight.T, n=n, m=m, vblk=16384)
    out = _sc_embed(x_flat, w_pad, bsz=bsz, hist=hist, m=m, rows_blk=4)
    return out.reshape(bsz, hist, m)


# speedup vs baseline: 2.6693x; 1.0137x over previous
import dataclasses
import functools

import jax
import jax.numpy as jnp
from jax import lax
from jax.experimental import pallas as pl
from jax.experimental.pallas import tpu as pltpu
from jax.experimental.pallas import tpu_sc as plsc

MAX_NORM = 1.0 - 0.0001
MAX_NORM_SQ = MAX_NORM * MAX_NORM
LANES = 16
GATHER_W = 128
NUM_CORES = 2
NUM_SUBCORES = 16
NUM_WORKERS = NUM_CORES * NUM_SUBCORES


def _pad_rows(wt, *, n, m, vblk):
    def body(x_ref, o_ref):
        o_ref[:, :m] = x_ref[...].T
        o_ref[:, m:] = jnp.zeros((vblk, GATHER_W - m), jnp.float32)

    return pl.pallas_call(
        body,
        grid=((n + vblk - 1) // vblk,),
        in_specs=[pl.BlockSpec((m, vblk), lambda i: (0, i))],
        out_specs=pl.BlockSpec((vblk, GATHER_W), lambda i: (i, 0)),
        out_shape=jax.ShapeDtypeStruct((n, GATHER_W), jnp.float32),
    )(wt)


@functools.partial(jax.jit, static_argnames=("bsz", "hist", "m", "rows_blk"))
def _sc_embed(x_flat, w_pad, *, bsz, hist, m, rows_blk):
    b = bsz * hist
    per_w = b // NUM_WORKERS
    chunk = rows_blk * hist
    n_chunks = per_w // chunk
    n_sub = m // LANES

    mesh = plsc.VectorSubcoreMesh(core_axis_name="c", subcore_axis_name="s")

    cparams = pltpu.CompilerParams()
    if "needs_layout_passes" in pltpu.CompilerParams.__dataclass_fields__:
        cparams = dataclasses.replace(cparams, needs_layout_passes=False)
    if "use_tc_tiling_on_sc" in pltpu.CompilerParams.__dataclass_fields__:
        cparams = dataclasses.replace(cparams, use_tc_tiling_on_sc=True)

    @functools.partial(
        pl.kernel,
        out_type=jax.ShapeDtypeStruct((b, m), jnp.float32),
        mesh=mesh,
        compiler_params=cparams,
        scratch_types=[
            pltpu.VMEM((chunk,), jnp.int32),
            pltpu.VMEM((chunk,), jnp.int32),
            pltpu.VMEM((chunk, GATHER_W), jnp.float32),
            pltpu.VMEM((chunk, GATHER_W), jnp.float32),
            pltpu.VMEM((chunk, m), jnp.float32),
            pltpu.SemaphoreType.DMA,
            pltpu.SemaphoreType.DMA,
        ],
    )
    def body(
        x_hbm, w_hbm, out_hbm, idx_a, idx_b, gbuf_a, gbuf_b, stage, sem0, sem1
    ):
        wid = lax.axis_index("s") * NUM_CORES + lax.axis_index("c")
        base = wid * per_w
        idxs = (idx_a, idx_b)
        gbufs = (gbuf_a, gbuf_b)
        sems = (sem0, sem1)

        def fetch(k, buf):
            off = base + k * chunk
            pltpu.sync_copy(x_hbm.at[pl.ds(off, chunk)], idxs[buf])
            pltpu.async_copy(w_hbm.at[idxs[buf]], gbufs[buf], sems[buf])

        def process(k, phase):
            off = base + k * chunk
            gbuf = gbufs[phase]
            pltpu.make_async_copy(
                w_hbm.at[pl.ds(0, chunk)], gbuf, sems[phase]
            ).wait()

            @pl.when(k + 1 < n_chunks)
            def _():
                fetch(k + 1, 1 - phase)

            def repack_row(q, gmax):
                p = jnp.zeros((LANES,), jnp.float32)
                for j in range(n_sub):
                    v = gbuf[q, pl.ds(j * LANES, LANES)]
                    stage[q, pl.ds(j * LANES, LANES)] = v
                    p = p + v * v
                return jnp.maximum(gmax, p)

            gmax = lax.fori_loop(
                0, chunk, repack_row, jnp.zeros((LANES,), jnp.float32)
            )
            bound = jnp.sum(gmax)

            @pl.when(bound > MAX_NORM_SQ)
            def _():
                def fix_row(q, carry):
                    vs = [
                        stage[q, pl.ds(j * LANES, LANES)]
                        for j in range(n_sub)
                    ]
                    p = jnp.zeros((LANES,), jnp.float32)
                    for v in vs:
                        p = p + v * v
                    s2 = jnp.sum(p)
                    s2v = lax.broadcast(s2, (LANES,))
                    bits = lax.bitcast_convert_type(s2v, jnp.int32)
                    y = lax.bitcast_convert_type(
                        0x5F3759DF - (bits >> 1), jnp.float32
                    )
                    for _ in range(3):
                        y = y * (1.5 - 0.5 * s2v * y * y)
                    norm = s2v * y
                    scale = jnp.where(
                        s2v > MAX_NORM_SQ,
                        MAX_NORM / (norm + 1e-7),
                        jnp.float32(1.0),
                    )
                    for j, v in enumerate(vs):
                        stage[q, pl.ds(j * LANES, LANES)] = v * scale
                    return carry

                lax.fori_loop(0, chunk, fix_row, 0)

            pltpu.sync_copy(stage, out_hbm.at[pl.ds(off, chunk)])

        fetch(0, 0)

        @pl.loop(0, n_chunks // 2)
        def _(kk):
            process(kk * 2, 0)
            process(kk * 2 + 1, 1)

    return body(x_flat, w_pad)


def kernel(x, weight):
    bsz, hist = x.shape
    n, m = weight.shape
    x_flat = x.reshape(bsz * hist).astype(jnp.int32)
    w_pad = _pad_rows(weight.T, n=n, m=m, vblk=32768)
    out = _sc_embed(x_flat, w_pad, bsz=bsz, hist=hist, m=m, rows_blk=4)
    return out.reshape(bsz, hist, m)


# speedup vs baseline: 3.0294x; 1.1349x over previous
import dataclasses
import functools

import jax
import jax.numpy as jnp
from jax import lax
from jax.experimental import pallas as pl
from jax.experimental.pallas import tpu as pltpu
from jax.experimental.pallas import tpu_sc as plsc

MAX_NORM = 1.0 - 0.0001
MAX_NORM_SQ = MAX_NORM * MAX_NORM
LANES = 16
GATHER_W = 128
NUM_CORES = 2
NUM_SUBCORES = 16
NUM_WORKERS = NUM_CORES * NUM_SUBCORES


def _pad_rows(wt, *, n, m, vblk):
    def body(x_ref, o_ref):
        o_ref[:, :m] = x_ref[...].T
        o_ref[:, m:] = jnp.zeros((vblk, GATHER_W - m), jnp.float32)

    return pl.pallas_call(
        body,
        grid=((n + vblk - 1) // vblk,),
        in_specs=[pl.BlockSpec((m, vblk), lambda i: (0, i))],
        out_specs=pl.BlockSpec((vblk, GATHER_W), lambda i: (i, 0)),
        out_shape=jax.ShapeDtypeStruct((n, GATHER_W), jnp.float32),
    )(wt)


@functools.partial(jax.jit, static_argnames=("bsz", "hist", "m", "rows_blk"))
def _sc_embed(x_flat, w_pad, *, bsz, hist, m, rows_blk):
    b = bsz * hist
    per_w = b // NUM_WORKERS
    chunk = rows_blk * hist
    n_chunks = per_w // chunk
    n_sub = m // LANES

    mesh = plsc.VectorSubcoreMesh(core_axis_name="c", subcore_axis_name="s")

    cparams = pltpu.CompilerParams()
    if "needs_layout_passes" in pltpu.CompilerParams.__dataclass_fields__:
        cparams = dataclasses.replace(cparams, needs_layout_passes=False)
    if "use_tc_tiling_on_sc" in pltpu.CompilerParams.__dataclass_fields__:
        cparams = dataclasses.replace(cparams, use_tc_tiling_on_sc=True)

    @functools.partial(
        pl.kernel,
        out_type=jax.ShapeDtypeStruct((b, m), jnp.float32),
        mesh=mesh,
        compiler_params=cparams,
        scratch_types=[
            pltpu.VMEM((chunk,), jnp.int32),
            pltpu.VMEM((chunk,), jnp.int32),
            pltpu.VMEM((chunk, GATHER_W), jnp.float32),
            pltpu.VMEM((chunk, GATHER_W), jnp.float32),
            pltpu.VMEM((chunk, m), jnp.float32),
            pltpu.SemaphoreType.DMA,
            pltpu.SemaphoreType.DMA,
        ],
    )
    def body(
        x_hbm, w_hbm, out_hbm, idx_a, idx_b, gbuf_a, gbuf_b, stage, sem0, sem1
    ):
        wid = lax.axis_index("s") * NUM_CORES + lax.axis_index("c")
        base = wid * per_w
        idxs = (idx_a, idx_b)
        gbufs = (gbuf_a, gbuf_b)
        sems = (sem0, sem1)

        def fetch(k, buf):
            off = base + k * chunk
            pltpu.sync_copy(x_hbm.at[pl.ds(off, chunk)], idxs[buf])
            pltpu.async_copy(w_hbm.at[idxs[buf]], gbufs[buf], sems[buf])

        def process(k, phase):
            off = base + k * chunk
            gbuf = gbufs[phase]
            pltpu.make_async_copy(
                w_hbm.at[pl.ds(0, chunk)], gbuf, sems[phase]
            ).wait()

            @pl.when(k + 1 < n_chunks)
            def _():
                fetch(k + 1, 1 - phase)

            def repack_row(q, gmax):
                p = jnp.zeros((LANES,), jnp.float32)
                for j in range(n_sub):
                    v = gbuf[q, pl.ds(j * LANES, LANES)]
                    stage[q, pl.ds(j * LANES, LANES)] = v
                    p = p + v * v
                return jnp.maximum(gmax, p)

            gmax = lax.fori_loop(
                0, chunk, repack_row, jnp.zeros((LANES,), jnp.float32)
            )
            bound = jnp.sum(gmax)

            @pl.when(bound > MAX_NORM_SQ)
            def _():
                def fix_row(q, carry):
                    vs = [
                        stage[q, pl.ds(j * LANES, LANES)]
                        for j in range(n_sub)
                    ]
                    p = jnp.zeros((LANES,), jnp.float32)
                    for v in vs:
                        p = p + v * v
                    s2 = jnp.sum(p)
                    s2v = lax.broadcast(s2, (LANES,))
                    bits = lax.bitcast_convert_type(s2v, jnp.int32)
                    y = lax.bitcast_convert_type(
                        0x5F3759DF - (bits >> 1), jnp.float32
                    )
                    for _ in range(3):
                        y = y * (1.5 - 0.5 * s2v * y * y)
                    norm = s2v * y
                    scale = jnp.where(
                        s2v > MAX_NORM_SQ,
                        MAX_NORM / (norm + 1e-7),
                        jnp.float32(1.0),
                    )
                    for j, v in enumerate(vs):
                        stage[q, pl.ds(j * LANES, LANES)] = v * scale
                    return carry

                lax.fori_loop(0, chunk, fix_row, 0)

            pltpu.sync_copy(stage, out_hbm.at[pl.ds(off, chunk)])

        fetch(0, 0)

        @pl.loop(0, n_chunks // 2)
        def _(kk):
            process(kk * 2, 0)
            process(kk * 2 + 1, 1)

    return body(x_flat, w_pad)


def _to_batch_minor(out2d, *, bsz, hist, m, bblk):
    def body(x_ref, o_ref):
        o_ref[...] = x_ref[...].reshape(bblk, hist, m).transpose(1, 2, 0)

    return pl.pallas_call(
        body,
        grid=(bsz // bblk,),
        in_specs=[pl.BlockSpec((bblk * hist, m), lambda i: (i, 0))],
        out_specs=pl.BlockSpec((hist, m, bblk), lambda i: (0, 0, i)),
        out_shape=jax.ShapeDtypeStruct((hist, m, bsz), jnp.float32),
    )(out2d)


def kernel(x, weight):
    bsz, hist = x.shape
    n, m = weight.shape
    x_flat = x.reshape(bsz * hist).astype(jnp.int32)
    w_pad = _pad_rows(weight.T, n=n, m=m, vblk=32768)
    out = _sc_embed(x_flat, w_pad, bsz=bsz, hist=hist, m=m, rows_blk=4)
    out_t = _to_batch_minor(out, bsz=bsz, hist=hist, m=m, bblk=128)
    return out_t.transpose(2, 0, 1)
